# baseline (speedup 1.0000x reference)
import functools

import jax
import jax.numpy as jnp
from jax import lax
from jax.experimental import pallas as pl
from jax.experimental.pallas import tpu as pltpu
from jax.experimental.pallas import tpu_sc as plsc

N_NODES = 10000
NP = 10240
D = 128
DH = D // 2
E = 320000
CH = 128
NCH = 160
IBLK = 16
NBLK = NCH // IBLK
EP = 16 * NCH * CH
RPT = NP // 16


def _make_agg(with_hist, table_rows):
    mesh = plsc.VectorSubcoreMesh(core_axis_name="c", subcore_axis_name="s")
    tpr = table_rows // 16
    out_type = [jax.ShapeDtypeStruct((2, NP, DH), jnp.float32)]
    scratch = [
        pltpu.MemorySpace.VMEM_SHARED((table_rows, DH), jnp.float32),
        pltpu.MemorySpace.VMEM_SHARED((NP, DH), jnp.float32),
        pltpu.VMEM((IBLK, CH), jnp.int32),
        pltpu.VMEM((IBLK, CH), jnp.int32),
        pltpu.VMEM((2, CH, DH), jnp.float32),
        pltpu.SemaphoreType.DMA,
        pltpu.SemaphoreType.DMA,
    ]
    if with_hist:
        out_type.append(jax.ShapeDtypeStruct((2, 16, NP), jnp.float32))
        scratch.append(pltpu.VMEM((NP,), jnp.float32))

    @functools.partial(
        pl.kernel,
        out_type=out_type,
        mesh=mesh,
        compiler_params=pltpu.CompilerParams(needs_layout_passes=False,
                                             use_tc_tiling_on_sc=False),
        scratch_types=scratch,
    )
    def agg(table_h, srcs_h, dsts_h, zd_h, sums_h, *rest):
        if with_hist:
            cnts_h, table_sp, accum, srcv, dstv, rows, gsem, ssem, hist = rest
        else:
            table_sp, accum, srcv, dstv, rows, gsem, ssem = rest
        c = lax.axis_index("c")
        s = lax.axis_index("s")
        base = s * RPT
        ones16 = jnp.ones((16,), jnp.float32)
        zeros16 = jnp.zeros((16,), jnp.float32)
        pltpu.sync_copy(table_h.at[pl.ds(s * tpr, tpr)],
                        table_sp.at[pl.ds(s * tpr, tpr)])
        pltpu.sync_copy(zd_h, accum.at[pl.ds(base, RPT)])

        if with_hist:
            def zstep(g, carry):
                hist[pl.ds(g * 16, 16)] = zeros16
                return carry

            lax.fori_loop(0, NP // 16, zstep, 0)
        plsc.subcore_barrier()

        def blk(bk, carry):
            pltpu.sync_copy(srcs_h.at[c, s, pl.ds(bk * IBLK, IBLK)], srcv)
            pltpu.sync_copy(dsts_h.at[c, s, pl.ds(bk * IBLK, IBLK)], dstv)
            gat = [None] * IBLK
            sct = [None] * IBLK
            gat[0] = pltpu.async_copy(table_sp.at[srcv.at[0]], rows.at[0], gsem)
            for g in range(IBLK):
                b = g % 2
                if g >= 1:
                    sct[g - 1].wait()
                if g + 1 < IBLK:
                    gat[g + 1] = pltpu.async_copy(
                        table_sp.at[srcv.at[g + 1]], rows.at[1 - b], gsem)
                gat[g].wait()
                sct[g] = pltpu.async_copy(
                    rows.at[b], accum.at[dstv.at[g]], ssem, add=True)
                if with_hist:
                    for j in range(CH // 16):
                        idx16 = dstv[g, pl.ds(j * 16, 16)]
                        plsc.addupdate_scatter(hist, [idx16], ones16)
            sct[IBLK - 1].wait()
            return carry

        lax.fori_loop(0, NBLK, blk, 0)
        plsc.subcore_barrier()
        pltpu.sync_copy(accum.at[pl.ds(base, RPT)],
                        sums_h.at[c, pl.ds(base, RPT)])
        if with_hist:
            pltpu.sync_copy(hist, cnts_h.at[c, s])

    return agg


_agg_hist_l1 = _make_agg(True, N_NODES)
_agg_nohist_l1 = _make_agg(False, N_NODES)
_agg_nohist_l2 = _make_agg(False, NP)


def _tc_layer(s0a, s0b, s1a, s1b, c0, c1, W0, b0, W1, b1, relu, split_out):
    BLK = 1024

    def body(s0a_r, s0b_r, s1a_r, s1b_r, c0_r, c1_r,
             W0_r, b0_r, W1_r, b1_r, *outs):
        c0b = jnp.sum(c0_r[...], axis=1, keepdims=True)
        c1b = jnp.sum(c1_r[...], axis=1, keepdims=True)
        s0 = jnp.concatenate([s0a_r[...], s0b_r[...]], axis=1)
        s1 = jnp.concatenate([s1a_r[...], s1b_r[...]], axis=1)
        m0 = s0 / jnp.maximum(c0b, 1.0)
        m1 = s1 / jnp.maximum(c1b, 1.0)
        acc = jnp.dot(m0, W0_r[...], preferred_element_type=jnp.float32)
        acc = acc + jnp.dot(m1, W1_r[...], preferred_element_type=jnp.float32)
        acc = acc + jnp.where(c0b > 0.0, 1.0, 0.0) * b0_r[...]
        acc = acc + jnp.where(c1b > 0.0, 1.0, 0.0) * b1_r[...]
        if relu:
            acc = jnp.maximum(acc, 0.0)
        if split_out:
            outs[0][...] = acc[:, :DH]
            outs[1][...] = acc[:, DH:]
        else:
            outs[0][...] = acc

    half = pl.BlockSpec((BLK, DH), lambda i: (i, 0))
    if split_out:
        out_specs = [half, half]
        out_shape = [jax.ShapeDtypeStruct((NP, DH), jnp.float32)] * 2
    else:
        out_specs = [pl.BlockSpec((BLK, D), lambda i: (i, 0))]
        out_shape = [jax.ShapeDtypeStruct((NP, D), jnp.float32)]
    return pl.pallas_call(
        body,
        grid=(NP // BLK,),
        in_specs=[
            half, half, half, half,
            pl.BlockSpec((BLK, 16), lambda i: (i, 0)),
            pl.BlockSpec((BLK, 16), lambda i: (i, 0)),
            pl.BlockSpec((D, D), lambda i: (0, 0)),
            pl.BlockSpec((1, D), lambda i: (0, 0)),
            pl.BlockSpec((D, D), lambda i: (0, 0)),
            pl.BlockSpec((1, D), lambda i: (0, 0)),
        ],
        out_specs=out_specs,
        out_shape=out_shape,
    )(s0a, s0b, s1a, s1b, c0, c1, W0, b0, W1, b1)


def kernel(feat, edge_index_rel0, edge_index_rel1,
           W1_rel0, b1_rel0, W1_rel1, b1_rel1,
           W2_rel0, b2_rel0, W2_rel1, b2_rel1):
    ei0 = edge_index_rel0.astype(jnp.int32)
    ei1 = edge_index_rel1.astype(jnp.int32)

    def prep(ei):
        src = jnp.concatenate([ei[0], jnp.zeros((EP - E,), jnp.int32)])
        dst = jnp.concatenate([ei[1], jnp.full((EP - E,), N_NODES, jnp.int32)])
        return src.reshape(16, NCH, CH), dst.reshape(16, NCH, CH)

    s0, d0 = prep(ei0)
    s1, d1 = prep(ei1)
    srcs = jnp.stack([s0, s1])
    dsts = jnp.stack([d0, d1])
    zeros_d = jnp.zeros((RPT, DH), jnp.float32)

    featA = feat[:, :DH]
    featB = feat[:, DH:]
    sumsA, cnts = _agg_hist_l1(featA, srcs, dsts, zeros_d)
    (sumsB,) = _agg_nohist_l1(featB, srcs, dsts, zeros_d)
    c0 = cnts[0].T
    c1 = cnts[1].T
    hA, hB = _tc_layer(sumsA[0], sumsB[0], sumsA[1], sumsB[1], c0, c1,
                       W1_rel0, b1_rel0.reshape(1, D),
                       W1_rel1, b1_rel1.reshape(1, D),
                       relu=True, split_out=True)
    (sums2A,) = _agg_nohist_l2(hA, srcs, dsts, zeros_d)
    (sums2B,) = _agg_nohist_l2(hB, srcs, dsts, zeros_d)
    (out,) = _tc_layer(sums2A[0], sums2B[0], sums2A[1], sums2B[1], c0, c1,
                       W2_rel0, b2_rel0.reshape(1, D),
                       W2_rel1, b2_rel1.reshape(1, D),
                       relu=False, split_out=False)
    return out[:N_NODES]


# speedup vs baseline: 1.1745x; 1.1745x over previous
import functools

import jax
import jax.numpy as jnp
from jax import lax
from jax.experimental import pallas as pl
from jax.experimental.pallas import tpu as pltpu
from jax.experimental.pallas import tpu_sc as plsc

N_NODES = 10000
NP = 10240
D = 128
DH = D // 2
E = 320000
CH = 128
NCH = 160
IBLK = 16
NBLK = NCH // IBLK
EP = 16 * NCH * CH
NBUF = 4
RPT = NP // 16


def _make_agg(with_hist, table_rows):
    mesh = plsc.VectorSubcoreMesh(core_axis_name="c", subcore_axis_name="s")
    tpr = table_rows // 16
    out_type = [jax.ShapeDtypeStruct((2, NP, DH), jnp.float32)]
    scratch = [
        pltpu.MemorySpace.VMEM_SHARED((table_rows, DH), jnp.float32),
        pltpu.MemorySpace.VMEM_SHARED((NP, DH), jnp.float32),
        pltpu.VMEM((IBLK, CH), jnp.int32),
        pltpu.VMEM((IBLK, CH), jnp.int32),
        pltpu.VMEM((NBUF, CH, DH), jnp.float32),
        pltpu.SemaphoreType.DMA,
        pltpu.SemaphoreType.DMA,
    ]
    if with_hist:
        out_type.append(jax.ShapeDtypeStruct((2, 16, NP), jnp.float32))
        scratch.append(pltpu.VMEM((NP,), jnp.float32))

    @functools.partial(
        pl.kernel,
        out_type=out_type,
        mesh=mesh,
        compiler_params=pltpu.CompilerParams(needs_layout_passes=False,
                                             use_tc_tiling_on_sc=False),
        scratch_types=scratch,
    )
    def agg(table_h, srcs_h, dsts_h, zd_h, sums_h, *rest):
        if with_hist:
            cnts_h, table_sp, accum, srcv, dstv, rows, gsem, ssem, hist = rest
        else:
            table_sp, accum, srcv, dstv, rows, gsem, ssem = rest
        c = lax.axis_index("c")
        s = lax.axis_index("s")
        base = s * RPT
        ones16 = jnp.ones((16,), jnp.float32)
        zeros16 = jnp.zeros((16,), jnp.float32)
        pltpu.sync_copy(table_h.at[pl.ds(s * tpr, tpr)],
                        table_sp.at[pl.ds(s * tpr, tpr)])
        pltpu.sync_copy(zd_h, accum.at[pl.ds(base, RPT)])

        if with_hist:
            def zstep(g, carry):
                hist[pl.ds(g * 16, 16)] = zeros16
                return carry

            lax.fori_loop(0, NP // 16, zstep, 0)
        plsc.subcore_barrier()

        def blk(bk, carry):
            pltpu.sync_copy(srcs_h.at[c, s, pl.ds(bk * IBLK, IBLK)], srcv)
            pltpu.sync_copy(dsts_h.at[c, s, pl.ds(bk * IBLK, IBLK)], dstv)
            gat = [None] * IBLK
            sct = [None] * IBLK
            gat[0] = pltpu.async_copy(table_sp.at[srcv.at[0]], rows.at[0], gsem)
            for g in range(IBLK):
                b = g % NBUF
                if g >= NBUF - 1:
                    sct[g - (NBUF - 1)].wait()
                if g + 1 < IBLK:
                    gat[g + 1] = pltpu.async_copy(
                        table_sp.at[srcv.at[g + 1]], rows.at[(g + 1) % NBUF],
                        gsem)
                gat[g].wait()
                sct[g] = pltpu.async_copy(
                    rows.at[b], accum.at[dstv.at[g]], ssem, add=True)
                if with_hist:
                    for j in range(CH // 16):
                        idx16 = dstv[g, pl.ds(j * 16, 16)]
                        plsc.addupdate_scatter(hist, [idx16], ones16)
            for t in range(max(0, IBLK - (NBUF - 1)), IBLK):
                sct[t].wait()
            return carry

        lax.fori_loop(0, NBLK, blk, 0)
        plsc.subcore_barrier()
        pltpu.sync_copy(accum.at[pl.ds(base, RPT)],
                        sums_h.at[c, pl.ds(base, RPT)])
        if with_hist:
            pltpu.sync_copy(hist, cnts_h.at[c, s])

    return agg


_agg_hist_l1 = _make_agg(True, N_NODES)
_agg_nohist_l1 = _make_agg(False, N_NODES)
_agg_nohist_l2 = _make_agg(False, NP)


def _tc_layer(s0a, s0b, s1a, s1b, c0, c1, W0, b0, W1, b1, relu, split_out):
    BLK = 1024

    def body(s0a_r, s0b_r, s1a_r, s1b_r, c0_r, c1_r,
             W0_r, b0_r, W1_r, b1_r, *outs):
        c0b = jnp.sum(c0_r[...], axis=1, keepdims=True)
        c1b = jnp.sum(c1_r[...], axis=1, keepdims=True)
        s0 = jnp.concatenate([s0a_r[...], s0b_r[...]], axis=1)
        s1 = jnp.concatenate([s1a_r[...], s1b_r[...]], axis=1)
        m0 = s0 / jnp.maximum(c0b, 1.0)
        m1 = s1 / jnp.maximum(c1b, 1.0)
        acc = jnp.dot(m0, W0_r[...], preferred_element_type=jnp.float32)
        acc = acc + jnp.dot(m1, W1_r[...], preferred_element_type=jnp.float32)
        acc = acc + jnp.where(c0b > 0.0, 1.0, 0.0) * b0_r[...]
        acc = acc + jnp.where(c1b > 0.0, 1.0, 0.0) * b1_r[...]
        if relu:
            acc = jnp.maximum(acc, 0.0)
        if split_out:
            outs[0][...] = acc[:, :DH]
            outs[1][...] = acc[:, DH:]
        else:
            outs[0][...] = acc

    half = pl.BlockSpec((BLK, DH), lambda i: (i, 0))
    if split_out:
        out_specs = [half, half]
        out_shape = [jax.ShapeDtypeStruct((NP, DH), jnp.float32)] * 2
    else:
        out_specs = [pl.BlockSpec((BLK, D), lambda i: (i, 0))]
        out_shape = [jax.ShapeDtypeStruct((NP, D), jnp.float32)]
    return pl.pallas_call(
        body,
        grid=(NP // BLK,),
        in_specs=[
            half, half, half, half,
            pl.BlockSpec((BLK, 16), lambda i: (i, 0)),
            pl.BlockSpec((BLK, 16), lambda i: (i, 0)),
            pl.BlockSpec((D, D), lambda i: (0, 0)),
            pl.BlockSpec((1, D), lambda i: (0, 0)),
            pl.BlockSpec((D, D), lambda i: (0, 0)),
            pl.BlockSpec((1, D), lambda i: (0, 0)),
        ],
        out_specs=out_specs,
        out_shape=out_shape,
    )(s0a, s0b, s1a, s1b, c0, c1, W0, b0, W1, b1)


def kernel(feat, edge_index_rel0, edge_index_rel1,
           W1_rel0, b1_rel0, W1_rel1, b1_rel1,
           W2_rel0, b2_rel0, W2_rel1, b2_rel1):
    ei0 = edge_index_rel0.astype(jnp.int32)
    ei1 = edge_index_rel1.astype(jnp.int32)

    def prep(ei):
        src = jnp.concatenate([ei[0], jnp.zeros((EP - E,), jnp.int32)])
        dst = jnp.concatenate([ei[1], jnp.full((EP - E,), N_NODES, jnp.int32)])
        return src.reshape(16, NCH, CH), dst.reshape(16, NCH, CH)

    s0, d0 = prep(ei0)
    s1, d1 = prep(ei1)
    srcs = jnp.stack([s0, s1])
    dsts = jnp.stack([d0, d1])
    zeros_d = jnp.zeros((RPT, DH), jnp.float32)

    featA = feat[:, :DH]
    featB = feat[:, DH:]
    sumsA, cnts = _agg_hist_l1(featA, srcs, dsts, zeros_d)
    (sumsB,) = _agg_nohist_l1(featB, srcs, dsts, zeros_d)
    c0 = cnts[0].T
    c1 = cnts[1].T
    hA, hB = _tc_layer(sumsA[0], sumsB[0], sumsA[1], sumsB[1], c0, c1,
                       W1_rel0, b1_rel0.reshape(1, D),
                       W1_rel1, b1_rel1.reshape(1, D),
                       relu=True, split_out=True)
    (sums2A,) = _agg_nohist_l2(hA, srcs, dsts, zeros_d)
    (sums2B,) = _agg_nohist_l2(hB, srcs, dsts, zeros_d)
    (out,) = _tc_layer(sums2A[0], sums2B[0], sums2A[1], sums2B[1], c0, c1,
                       W2_rel0, b2_rel0.reshape(1, D),
                       W2_rel1, b2_rel1.reshape(1, D),
                       relu=False, split_out=False)
    return out[:N_NODES]


# speedup vs baseline: 1.2868x; 1.0956x over previous
import functools

import jax
import jax.numpy as jnp
from jax import lax
from jax.experimental import pallas as pl
from jax.experimental.pallas import tpu as pltpu
from jax.experimental.pallas import tpu_sc as plsc

N_NODES = 10000
NP = 10240
D = 128
DH = D // 2
E = 320000
CH = 128
NCH = 160
IBLK = 16
NBLK = NCH // IBLK
EP = 16 * NCH * CH
RPT = NP // 16


def _make_agg(with_hist, table_rows, NBUF):
    mesh = plsc.VectorSubcoreMesh(core_axis_name="c", subcore_axis_name="s")
    tpr = table_rows // 16
    out_type = [jax.ShapeDtypeStruct((2, NP, DH), jnp.float32)]
    scratch = [
        pltpu.MemorySpace.VMEM_SHARED((table_rows, DH), jnp.float32),
        pltpu.MemorySpace.VMEM_SHARED((NP, DH), jnp.float32),
        pltpu.VMEM((2, IBLK, CH), jnp.int32),
        pltpu.VMEM((2, IBLK, CH), jnp.int32),
        pltpu.VMEM((NBUF, CH, DH), jnp.float32),
        pltpu.SemaphoreType.DMA,
        pltpu.SemaphoreType.DMA,
        pltpu.SemaphoreType.DMA,
    ]
    if with_hist:
        out_type.append(jax.ShapeDtypeStruct((2, 16, NP), jnp.float32))
        scratch.append(pltpu.VMEM((NP,), jnp.float32))

    @functools.partial(
        pl.kernel,
        out_type=out_type,
        mesh=mesh,
        compiler_params=pltpu.CompilerParams(needs_layout_passes=False,
                                             use_tc_tiling_on_sc=False),
        scratch_types=scratch,
    )
    def agg(table_h, srcs_h, dsts_h, zd_h, sums_h, *rest):
        if with_hist:
            (cnts_h, table_sp, accum, srcv, dstv, rows,
             gsem, ssem, isem, hist) = rest
        else:
            table_sp, accum, srcv, dstv, rows, gsem, ssem, isem = rest
        c = lax.axis_index("c")
        s = lax.axis_index("s")
        base = s * RPT
        ones16 = jnp.ones((16,), jnp.float32)
        zeros16 = jnp.zeros((16,), jnp.float32)
        pltpu.sync_copy(table_h.at[pl.ds(s * tpr, tpr)],
                        table_sp.at[pl.ds(s * tpr, tpr)])
        pltpu.sync_copy(zd_h, accum.at[pl.ds(base, RPT)])

        if with_hist:
            def zstep(g, carry):
                hist[pl.ds(g * 16, 16)] = zeros16
                return carry

            lax.fori_loop(0, NP // 16, zstep, 0)
        pltpu.sync_copy(srcs_h.at[c, s, pl.ds(0, IBLK)], srcv.at[0])
        pltpu.sync_copy(dsts_h.at[c, s, pl.ds(0, IBLK)], dstv.at[0])
        plsc.subcore_barrier()

        gat = [None] * NCH
        sct = [None] * NCH
        idxp = None
        gat[0] = pltpu.async_copy(
            table_sp.at[srcv.at[0].at[0]], rows.at[0], gsem)
        for g in range(NCH):
            bk, gl = divmod(g, IBLK)
            ib = bk % 2
            b = g % NBUF
            if g >= NBUF - 1:
                sct[g - (NBUF - 1)].wait()
            if gl == NBUF - 2 and bk + 1 < NBLK:
                idxp = (
                    pltpu.async_copy(
                        srcs_h.at[c, s, pl.ds((bk + 1) * IBLK, IBLK)],
                        srcv.at[1 - ib], isem),
                    pltpu.async_copy(
                        dsts_h.at[c, s, pl.ds((bk + 1) * IBLK, IBLK)],
                        dstv.at[1 - ib], isem),
                )
            if g + 1 < NCH:
                nbk, ngl = divmod(g + 1, IBLK)
                if ngl == 0:
                    idxp[0].wait()
                    idxp[1].wait()
                gat[g + 1] = pltpu.async_copy(
                    table_sp.at[srcv.at[nbk % 2].at[ngl]],
                    rows.at[(g + 1) % NBUF], gsem)
            gat[g].wait()
            sct[g] = pltpu.async_copy(
                rows.at[b], accum.at[dstv.at[ib].at[gl]], ssem, add=True)
            if with_hist:
                for j in range(CH // 16):
                    idx16 = dstv[ib, gl, pl.ds(j * 16, 16)]
                    plsc.addupdate_scatter(hist, [idx16], ones16)
        for t in range(max(0, NCH - (NBUF - 1)), NCH):
            sct[t].wait()
        plsc.subcore_barrier()
        pltpu.sync_copy(accum.at[pl.ds(base, RPT)],
                        sums_h.at[c, pl.ds(base, RPT)])
        if with_hist:
            pltpu.sync_copy(hist, cnts_h.at[c, s])

    return agg


_agg_hist_l1 = _make_agg(True, N_NODES, 3)
_agg_nohist_l1 = _make_agg(False, N_NODES, 4)
_agg_nohist_l2 = _make_agg(False, NP, 4)


def _tc_layer(s0a, s0b, s1a, s1b, c0, c1, W0, b0, W1, b1, relu, split_out):
    BLK = 1024

    def body(s0a_r, s0b_r, s1a_r, s1b_r, c0_r, c1_r,
             W0_r, b0_r, W1_r, b1_r, *outs):
        c0b = jnp.sum(c0_r[...], axis=1, keepdims=True)
        c1b = jnp.sum(c1_r[...], axis=1, keepdims=True)
        s0 = jnp.concatenate([s0a_r[...], s0b_r[...]], axis=1)
        s1 = jnp.concatenate([s1a_r[...], s1b_r[...]], axis=1)
        m0 = s0 / jnp.maximum(c0b, 1.0)
        m1 = s1 / jnp.maximum(c1b, 1.0)
        acc = jnp.dot(m0, W0_r[...], preferred_element_type=jnp.float32)
        acc = acc + jnp.dot(m1, W1_r[...], preferred_element_type=jnp.float32)
        acc = acc + jnp.where(c0b > 0.0, 1.0, 0.0) * b0_r[...]
        acc = acc + jnp.where(c1b > 0.0, 1.0, 0.0) * b1_r[...]
        if relu:
            acc = jnp.maximum(acc, 0.0)
        if split_out:
            outs[0][...] = acc[:, :DH]
            outs[1][...] = acc[:, DH:]
        else:
            outs[0][...] = acc

    half = pl.BlockSpec((BLK, DH), lambda i: (i, 0))
    if split_out:
        out_specs = [half, half]
        out_shape = [jax.ShapeDtypeStruct((NP, DH), jnp.float32)] * 2
    else:
        out_specs = [pl.BlockSpec((BLK, D), lambda i: (i, 0))]
        out_shape = [jax.ShapeDtypeStruct((NP, D), jnp.float32)]
    return pl.pallas_call(
        body,
        grid=(NP // BLK,),
        in_specs=[
            half, half, half, half,
            pl.BlockSpec((BLK, 16), lambda i: (i, 0)),
            pl.BlockSpec((BLK, 16), lambda i: (i, 0)),
            pl.BlockSpec((D, D), lambda i: (0, 0)),
            pl.BlockSpec((1, D), lambda i: (0, 0)),
            pl.BlockSpec((D, D), lambda i: (0, 0)),
            pl.BlockSpec((1, D), lambda i: (0, 0)),
        ],
        out_specs=out_specs,
        out_shape=out_shape,
    )(s0a, s0b, s1a, s1b, c0, c1, W0, b0, W1, b1)


def kernel(feat, edge_index_rel0, edge_index_rel1,
           W1_rel0, b1_rel0, W1_rel1, b1_rel1,
           W2_rel0, b2_rel0, W2_rel1, b2_rel1):
    ei0 = edge_index_rel0.astype(jnp.int32)
    ei1 = edge_index_rel1.astype(jnp.int32)

    def prep(ei):
        src = jnp.concatenate([ei[0], jnp.zeros((EP - E,), jnp.int32)])
        dst = jnp.concatenate([ei[1], jnp.full((EP - E,), N_NODES, jnp.int32)])
        return src.reshape(16, NCH, CH), dst.reshape(16, NCH, CH)

    s0, d0 = prep(ei0)
    s1, d1 = prep(ei1)
    srcs = jnp.stack([s0, s1])
    dsts = jnp.stack([d0, d1])
    zeros_d = jnp.zeros((RPT, DH), jnp.float32)

    featA = feat[:, :DH]
    featB = feat[:, DH:]
    sumsA, cnts = _agg_hist_l1(featA, srcs, dsts, zeros_d)
    (sumsB,) = _agg_nohist_l1(featB, srcs, dsts, zeros_d)
    c0 = cnts[0].T
    c1 = cnts[1].T
    hA, hB = _tc_layer(sumsA[0], sumsB[0], sumsA[1], sumsB[1], c0, c1,
                       W1_rel0, b1_rel0.reshape(1, D),
                       W1_rel1, b1_rel1.reshape(1, D),
                       relu=True, split_out=True)
    (sums2A,) = _agg_nohist_l2(hA, srcs, dsts, zeros_d)
    (sums2B,) = _agg_nohist_l2(hB, srcs, dsts, zeros_d)
    (out,) = _tc_layer(sums2A[0], sums2B[0], sums2A[1], sums2B[1], c0, c1,
                       W2_rel0, b2_rel0.reshape(1, D),
                       W2_rel1, b2_rel1.reshape(1, D),
                       relu=False, split_out=False)
    return out[:N_NODES]
